# baseline (device time: 64884 ns/iter reference)
import jax
import jax.numpy as jnp
from jax import lax
from jax.experimental import pallas as pl
from jax.experimental.pallas import tpu as pltpu

N_DEV = 4
B = 4
SQ = 256
D = 1024
HQ = 8
HKV = 2
DH = 128
SKV = 1024
SCALE = 0.08838834764831843
NBH = B * HQ


def kernel(x, Wq, Wo, K_ext, V_ext):
    def body(x_ref, wq_ref, wo_ref, k_hbm, v_hbm, out_ref,
             loc_o, loc_ml, rs_o, rs_ml, q2_ref, kv_ref, pt_ref,
             wqb_ref, wob_ref, kvb_ref,
             rso_send, rso_recv, rsml_send, rsml_recv,
             ag_send, ag_recv,
             rso_send2, rso_recv2, rsml_send2, rsml_recv2,
             ag_send2, ag_recv2, copy_sem, agb_ref, ones_ref):
        my = lax.axis_index("i")
        left = (my + N_DEV - 1) % N_DEV
        right = (my + 1) % N_DEV

        barrier = pltpu.get_barrier_semaphore()
        for nbr in (left, right):
            pl.semaphore_signal(
                barrier, inc=1,
                device_id=(nbr,), device_id_type=pl.DeviceIdType.MESH,
            )

        for b in range(B):
            for g in range(HKV):
                pltpu.make_async_copy(
                    k_hbm.at[b, :, g, :], kv_ref.at[0, b * HKV + g], copy_sem,
                ).start()
                pltpu.make_async_copy(
                    v_hbm.at[b, :, g, :], kv_ref.at[1, b * HKV + g], copy_sem,
                ).start()
        for b in range(B):
            for g in range(HKV):
                pltpu.make_async_copy(
                    k_hbm.at[b, :, g, :], kv_ref.at[0, b * HKV + g], copy_sem,
                ).wait()
                pltpu.make_async_copy(
                    v_hbm.at[b, :, g, :], kv_ref.at[1, b * HKV + g], copy_sem,
                ).wait()

        wqb_ref[...] = wq_ref[...].astype(jnp.bfloat16)
        wob_ref[...] = wo_ref[...].astype(jnp.bfloat16)
        ones_ref[...] = jnp.ones((8, SKV), jnp.bfloat16)
        for t in range(2):
            for i in range(B * HKV):
                kvb_ref[t, i] = kv_ref[t, i].astype(jnp.bfloat16)

        def flash_batch(bj):
            qb = lax.dot_general(
                x_ref[bj].astype(jnp.bfloat16), wqb_ref[...],
                (((1,), (0,)), ((), ())),
                preferred_element_type=jnp.float32,
            ) * SCALE
            for h in range(HQ):
                q2_ref[bj * HQ + h] = qb[:, h * DH:(h + 1) * DH].astype(
                    jnp.bfloat16)

            def step(h, carry):
                c = bj * HQ + h
                kvi = bj * HKV + h // (HQ // HKV)
                qh = q2_ref[c]
                kh = kvb_ref[0, kvi]
                vh = kvb_ref[1, kvi]
                st = lax.dot_general(
                    kh, qh, (((1,), (1,)), ((), ())),
                    preferred_element_type=jnp.float32,
                )
                pt_ref[...] = jnp.exp(st).astype(jnp.bfloat16)
                ot = lax.dot_general(
                    vh, pt_ref[...], (((0,), (0,)), ((), ())),
                    preferred_element_type=jnp.float32,
                )
                lb = lax.dot_general(
                    ones_ref[...], pt_ref[...], (((1,), (0,)), ((), ())),
                    preferred_element_type=jnp.float32,
                )
                loc_o[c] = ot.astype(jnp.bfloat16)
                loc_ml[c, 0:1, :] = lb[0:1, :]
                return carry

            lax.fori_loop(0, HQ, step, None)

        HH = HQ // 2

        def rs_rdmas(h, src_o, src_ml, go_right):
            if go_right:
                dst_o, dst_ml = rs_o.at[h, pl.ds(0, HH)], rs_ml.at[h, pl.ds(0, HH)]
                sems = (rso_send, rso_recv, rsml_send, rsml_recv)
                dev = right
            else:
                dst_o, dst_ml = rs_o.at[h, pl.ds(HH, HH)], rs_ml.at[h, pl.ds(HH, HH)]
                sems = (rso_send2, rso_recv2, rsml_send2, rsml_recv2)
                dev = left
            ro = pltpu.make_async_remote_copy(
                src_ref=src_o, dst_ref=dst_o,
                send_sem=sems[0].at[h], recv_sem=sems[1].at[h],
                device_id=(dev,), device_id_type=pl.DeviceIdType.MESH,
            )
            rml = pltpu.make_async_remote_copy(
                src_ref=src_ml, dst_ref=dst_ml,
                send_sem=sems[2].at[h], recv_sem=sems[3].at[h],
                device_id=(dev,), device_id_type=pl.DeviceIdType.MESH,
            )
            return ro, rml

        def merge_hop(h, go_right):
            if go_right:
                bm = (my + N_DEV - h - 1) % N_DEV
                k0 = 0
            else:
                bm = (my + h + 3) % N_DEV
                k0 = HH

            def step(i, carry):
                k = k0 + i
                c = bm * HQ + k
                rs_ml[h, k, 0:1, :] = rs_ml[h, k, 0:1, :] + loc_ml[c, 0:1, :]
                rs_o[h, k] = rs_o[h, k] + loc_o[c]
                return carry

            lax.fori_loop(0, HH, step, None)

        flash_batch(my)
        pl.semaphore_wait(barrier, 2)
        r0_o, r0_ml = rs_rdmas(
            0,
            loc_o.at[pl.ds(my * HQ, HH)],
            loc_ml.at[pl.ds(my * HQ, HH)],
            go_right=True,
        )
        r0_o.start()
        r0_ml.start()
        b2 = (my + 2) % N_DEV
        flash_batch(b2)
        l0_o, l0_ml = rs_rdmas(
            0,
            loc_o.at[pl.ds(b2 * HQ + HH, HH)],
            loc_ml.at[pl.ds(b2 * HQ + HH, HH)],
            go_right=False,
        )
        l0_o.start()
        l0_ml.start()
        flash_batch((my + 3) % N_DEV)

        r0_o.wait()
        r0_ml.wait()
        merge_hop(0, go_right=True)
        l0_o.wait()
        l0_ml.wait()
        merge_hop(0, go_right=False)
        for h in (1, 2):
            ro, rml = rs_rdmas(
                h, rs_o.at[h - 1, pl.ds(0, HH)], rs_ml.at[h - 1, pl.ds(0, HH)],
                go_right=True,
            )
            lo, lml = rs_rdmas(
                h, rs_o.at[h - 1, pl.ds(HH, HH)], rs_ml.at[h - 1, pl.ds(HH, HH)],
                go_right=False,
            )
            ro.start()
            lo.start()
            rml.start()
            lml.start()
            if h == 1:
                flash_batch((my + 1) % N_DEV)
            ro.wait()
            rml.wait()
            merge_hop(h, go_right=True)
            lo.wait()
            lml.wait()
            merge_hop(h, go_right=False)

        q = (my + 1) % N_DEV
        for k in range(HQ):
            ob = (
                rs_o[2, k].astype(jnp.float32) / rs_ml[2, k, 0:1, :]
            ).astype(jnp.bfloat16)
            term = lax.dot_general(
                ob, wob_ref[k * DH:(k + 1) * DH, :],
                (((0,), (0,)), ((), ())),
                preferred_element_type=jnp.float32,
            )
            if k == 0:
                out_ref[q] = term
            else:
                out_ref[q] = out_ref[q] + term

        agb_ref[q] = out_ref[q].astype(jnp.bfloat16)
        HSQ = SQ // 2
        for h in range(N_DEV - 1):
            sb_r = (q + N_DEV - h) % N_DEV
            sb_l = (q + h) % N_DEV
            ag_r = pltpu.make_async_remote_copy(
                src_ref=agb_ref.at[sb_r, pl.ds(0, HSQ)],
                dst_ref=agb_ref.at[sb_r, pl.ds(0, HSQ)],
                send_sem=ag_send.at[h], recv_sem=ag_recv.at[h],
                device_id=(right,), device_id_type=pl.DeviceIdType.MESH,
            )
            ag_l = pltpu.make_async_remote_copy(
                src_ref=agb_ref.at[sb_l, pl.ds(HSQ, HSQ)],
                dst_ref=agb_ref.at[sb_l, pl.ds(HSQ, HSQ)],
                send_sem=ag_send2.at[h], recv_sem=ag_recv2.at[h],
                device_id=(left,), device_id_type=pl.DeviceIdType.MESH,
            )
            ag_r.start()
            ag_l.start()
            ag_r.wait()
            ag_l.wait()
        for b in range(B):
            out_ref[b] = agb_ref[b].astype(jnp.float32)

    return pl.pallas_call(
        body,
        out_shape=jax.ShapeDtypeStruct((B, SQ, D), jnp.float32),
        in_specs=[
            pl.BlockSpec(memory_space=pltpu.MemorySpace.VMEM),
            pl.BlockSpec(memory_space=pltpu.MemorySpace.VMEM),
            pl.BlockSpec(memory_space=pltpu.MemorySpace.VMEM),
            pl.BlockSpec(memory_space=pltpu.MemorySpace.HBM),
            pl.BlockSpec(memory_space=pltpu.MemorySpace.HBM),
        ],
        out_specs=pl.BlockSpec(memory_space=pltpu.MemorySpace.VMEM),
        scratch_shapes=[
            pltpu.VMEM((NBH, DH, SQ), jnp.bfloat16),
            pltpu.VMEM((NBH, 1, SQ), jnp.float32),
            pltpu.VMEM((N_DEV - 1, HQ, DH, SQ), jnp.bfloat16),
            pltpu.VMEM((N_DEV - 1, HQ, 1, SQ), jnp.float32),
            pltpu.VMEM((NBH, SQ, DH), jnp.bfloat16),
            pltpu.VMEM((2, B * HKV, SKV, DH), jnp.float32),
            pltpu.VMEM((SKV, SQ), jnp.bfloat16),
            pltpu.VMEM((D, D), jnp.bfloat16),
            pltpu.VMEM((D, D), jnp.bfloat16),
            pltpu.VMEM((2, B * HKV, SKV, DH), jnp.bfloat16),
            pltpu.SemaphoreType.DMA((N_DEV - 1,)),
            pltpu.SemaphoreType.DMA((N_DEV - 1,)),
            pltpu.SemaphoreType.DMA((N_DEV - 1,)),
            pltpu.SemaphoreType.DMA((N_DEV - 1,)),
            pltpu.SemaphoreType.DMA((N_DEV - 1,)),
            pltpu.SemaphoreType.DMA((N_DEV - 1,)),
            pltpu.SemaphoreType.DMA((N_DEV - 1,)),
            pltpu.SemaphoreType.DMA((N_DEV - 1,)),
            pltpu.SemaphoreType.DMA((N_DEV - 1,)),
            pltpu.SemaphoreType.DMA((N_DEV - 1,)),
            pltpu.SemaphoreType.DMA((N_DEV - 1,)),
            pltpu.SemaphoreType.DMA((N_DEV - 1,)),
            pltpu.SemaphoreType.DMA,
            pltpu.VMEM((B, SQ, D), jnp.bfloat16),
            pltpu.VMEM((8, SKV), jnp.bfloat16),
        ],
        compiler_params=pltpu.CompilerParams(
            collective_id=0,
            vmem_limit_bytes=100 * 1024 * 1024,
        ),
    )(x, Wq, Wo, K_ext, V_ext)


# device time: 60674 ns/iter; 1.0694x vs baseline; 1.0694x over previous
import jax
import jax.numpy as jnp
from jax import lax
from jax.experimental import pallas as pl
from jax.experimental.pallas import tpu as pltpu

N_DEV = 4
B = 4
SQ = 256
D = 1024
HQ = 8
HKV = 2
DH = 128
SKV = 1024
SCALE = 0.08838834764831843
NBH = B * HQ


def kernel(x, Wq, Wo, K_ext, V_ext):
    def body(x_ref, wq_ref, wo_ref, k_hbm, v_hbm, out_ref,
             loc_o, loc_ml, rs_o, rs_ml, q2_ref, kv_ref, pt_ref,
             wqb_ref, wob_ref, kvb_ref,
             rso_send, rso_recv, rsml_send, rsml_recv,
             ag_send, ag_recv,
             rso_send2, rso_recv2, rsml_send2, rsml_recv2,
             ag_send2, ag_recv2, copy_sems, agb_ref):
        my = lax.axis_index("i")
        left = (my + N_DEV - 1) % N_DEV
        right = (my + 1) % N_DEV

        barrier = pltpu.get_barrier_semaphore()
        for nbr in (left, right):
            pl.semaphore_signal(
                barrier, inc=1,
                device_id=(nbr,), device_id_type=pl.DeviceIdType.MESH,
            )

        def kv_dmas(bj):
            ds = []
            for g in range(HKV):
                ds.append(pltpu.make_async_copy(
                    k_hbm.at[bj, :, g, :], kv_ref.at[0, bj * HKV + g],
                    copy_sems.at[bj],
                ))
                ds.append(pltpu.make_async_copy(
                    v_hbm.at[bj, :, g, :], kv_ref.at[1, bj * HKV + g],
                    copy_sems.at[bj],
                ))
            return ds

        for b in range(B):
            for d in kv_dmas(b):
                d.start()

        wqb_ref[...] = wq_ref[...].astype(jnp.bfloat16)
        wob_ref[...] = wo_ref[...].astype(jnp.bfloat16)

        def flash_batch(bj):
            for d in kv_dmas(bj):
                d.wait()
            for g in range(HKV):
                kvb_ref[0, bj * HKV + g] = kv_ref[0, bj * HKV + g].astype(
                    jnp.bfloat16)
                kvb_ref[1, bj * HKV + g] = kv_ref[1, bj * HKV + g].astype(
                    jnp.bfloat16)
            qb = lax.dot_general(
                x_ref[bj].astype(jnp.bfloat16), wqb_ref[...],
                (((1,), (0,)), ((), ())),
                preferred_element_type=jnp.float32,
            ) * SCALE
            for h in range(HQ):
                q2_ref[bj * HQ + h] = qb[:, h * DH:(h + 1) * DH].astype(
                    jnp.bfloat16)

            def step(h, carry):
                c = bj * HQ + h
                kvi = bj * HKV + h // (HQ // HKV)
                qh = q2_ref[c]
                kh = kvb_ref[0, kvi]
                vh = kvb_ref[1, kvi]
                st = lax.dot_general(
                    kh, qh, (((1,), (1,)), ((), ())),
                    preferred_element_type=jnp.float32,
                )
                e = jnp.exp(st)
                l_row = jnp.sum(e, axis=0, keepdims=True)
                pt_ref[...] = e.astype(jnp.bfloat16)
                ot = lax.dot_general(
                    vh, pt_ref[...], (((0,), (0,)), ((), ())),
                    preferred_element_type=jnp.float32,
                )
                loc_o[c] = ot.astype(jnp.bfloat16)
                loc_ml[c, 0:1, :] = l_row
                return carry

            lax.fori_loop(0, HQ, step, None)

        HH = HQ // 2

        def rs_rdmas(h, src_o, src_ml, go_right):
            if go_right:
                dst_o, dst_ml = rs_o.at[h, pl.ds(0, HH)], rs_ml.at[h, pl.ds(0, HH)]
                sems = (rso_send, rso_recv, rsml_send, rsml_recv)
                dev = right
            else:
                dst_o, dst_ml = rs_o.at[h, pl.ds(HH, HH)], rs_ml.at[h, pl.ds(HH, HH)]
                sems = (rso_send2, rso_recv2, rsml_send2, rsml_recv2)
                dev = left
            ro = pltpu.make_async_remote_copy(
                src_ref=src_o, dst_ref=dst_o,
                send_sem=sems[0].at[h], recv_sem=sems[1].at[h],
                device_id=(dev,), device_id_type=pl.DeviceIdType.MESH,
            )
            rml = pltpu.make_async_remote_copy(
                src_ref=src_ml, dst_ref=dst_ml,
                send_sem=sems[2].at[h], recv_sem=sems[3].at[h],
                device_id=(dev,), device_id_type=pl.DeviceIdType.MESH,
            )
            return ro, rml

        def merge_hop(h, go_right):
            if go_right:
                bm = (my + N_DEV - h - 1) % N_DEV
                k0 = 0
            else:
                bm = (my + h + 3) % N_DEV
                k0 = HH

            def step(i, carry):
                k = k0 + i
                c = bm * HQ + k
                rs_ml[h, k, 0:1, :] = rs_ml[h, k, 0:1, :] + loc_ml[c, 0:1, :]
                rs_o[h, k] = rs_o[h, k] + loc_o[c]
                return carry

            lax.fori_loop(0, HH, step, None)

        flash_batch(my)
        pl.semaphore_wait(barrier, 2)
        r0_o, r0_ml = rs_rdmas(
            0,
            loc_o.at[pl.ds(my * HQ, HH)],
            loc_ml.at[pl.ds(my * HQ, HH)],
            go_right=True,
        )
        r0_o.start()
        r0_ml.start()
        b2 = (my + 2) % N_DEV
        flash_batch(b2)
        l0_o, l0_ml = rs_rdmas(
            0,
            loc_o.at[pl.ds(b2 * HQ + HH, HH)],
            loc_ml.at[pl.ds(b2 * HQ + HH, HH)],
            go_right=False,
        )
        l0_o.start()
        l0_ml.start()
        flash_batch((my + 3) % N_DEV)

        r0_o.wait()
        r0_ml.wait()
        merge_hop(0, go_right=True)
        l0_o.wait()
        l0_ml.wait()
        merge_hop(0, go_right=False)
        for h in (1, 2):
            ro, rml = rs_rdmas(
                h, rs_o.at[h - 1, pl.ds(0, HH)], rs_ml.at[h - 1, pl.ds(0, HH)],
                go_right=True,
            )
            lo, lml = rs_rdmas(
                h, rs_o.at[h - 1, pl.ds(HH, HH)], rs_ml.at[h - 1, pl.ds(HH, HH)],
                go_right=False,
            )
            ro.start()
            lo.start()
            rml.start()
            lml.start()
            if h == 1:
                flash_batch((my + 1) % N_DEV)
            ro.wait()
            rml.wait()
            merge_hop(h, go_right=True)
            lo.wait()
            lml.wait()
            merge_hop(h, go_right=False)

        q = (my + 1) % N_DEV
        for k in range(HQ):
            ob = (
                rs_o[2, k].astype(jnp.float32) / rs_ml[2, k, 0:1, :]
            ).astype(jnp.bfloat16)
            term = lax.dot_general(
                ob, wob_ref[k * DH:(k + 1) * DH, :],
                (((0,), (0,)), ((), ())),
                preferred_element_type=jnp.float32,
            )
            if k == 0:
                out_ref[q] = term
            else:
                out_ref[q] = out_ref[q] + term

        agb_ref[q] = out_ref[q].astype(jnp.bfloat16)
        HSQ = SQ // 2
        for h in range(N_DEV - 1):
            sb_r = (q + N_DEV - h) % N_DEV
            sb_l = (q + h) % N_DEV
            ag_r = pltpu.make_async_remote_copy(
                src_ref=agb_ref.at[sb_r, pl.ds(0, HSQ)],
                dst_ref=agb_ref.at[sb_r, pl.ds(0, HSQ)],
                send_sem=ag_send.at[h], recv_sem=ag_recv.at[h],
                device_id=(right,), device_id_type=pl.DeviceIdType.MESH,
            )
            ag_l = pltpu.make_async_remote_copy(
                src_ref=agb_ref.at[sb_l, pl.ds(HSQ, HSQ)],
                dst_ref=agb_ref.at[sb_l, pl.ds(HSQ, HSQ)],
                send_sem=ag_send2.at[h], recv_sem=ag_recv2.at[h],
                device_id=(left,), device_id_type=pl.DeviceIdType.MESH,
            )
            ag_r.start()
            ag_l.start()
            ag_r.wait()
            ag_l.wait()
        for b in range(B):
            out_ref[b] = agb_ref[b].astype(jnp.float32)

    return pl.pallas_call(
        body,
        out_shape=jax.ShapeDtypeStruct((B, SQ, D), jnp.float32),
        in_specs=[
            pl.BlockSpec(memory_space=pltpu.MemorySpace.VMEM),
            pl.BlockSpec(memory_space=pltpu.MemorySpace.VMEM),
            pl.BlockSpec(memory_space=pltpu.MemorySpace.VMEM),
            pl.BlockSpec(memory_space=pltpu.MemorySpace.HBM),
            pl.BlockSpec(memory_space=pltpu.MemorySpace.HBM),
        ],
        out_specs=pl.BlockSpec(memory_space=pltpu.MemorySpace.VMEM),
        scratch_shapes=[
            pltpu.VMEM((NBH, DH, SQ), jnp.bfloat16),
            pltpu.VMEM((NBH, 1, SQ), jnp.float32),
            pltpu.VMEM((N_DEV - 1, HQ, DH, SQ), jnp.bfloat16),
            pltpu.VMEM((N_DEV - 1, HQ, 1, SQ), jnp.float32),
            pltpu.VMEM((NBH, SQ, DH), jnp.bfloat16),
            pltpu.VMEM((2, B * HKV, SKV, DH), jnp.float32),
            pltpu.VMEM((SKV, SQ), jnp.bfloat16),
            pltpu.VMEM((D, D), jnp.bfloat16),
            pltpu.VMEM((D, D), jnp.bfloat16),
            pltpu.VMEM((2, B * HKV, SKV, DH), jnp.bfloat16),
            pltpu.SemaphoreType.DMA((N_DEV - 1,)),
            pltpu.SemaphoreType.DMA((N_DEV - 1,)),
            pltpu.SemaphoreType.DMA((N_DEV - 1,)),
            pltpu.SemaphoreType.DMA((N_DEV - 1,)),
            pltpu.SemaphoreType.DMA((N_DEV - 1,)),
            pltpu.SemaphoreType.DMA((N_DEV - 1,)),
            pltpu.SemaphoreType.DMA((N_DEV - 1,)),
            pltpu.SemaphoreType.DMA((N_DEV - 1,)),
            pltpu.SemaphoreType.DMA((N_DEV - 1,)),
            pltpu.SemaphoreType.DMA((N_DEV - 1,)),
            pltpu.SemaphoreType.DMA((N_DEV - 1,)),
            pltpu.SemaphoreType.DMA((N_DEV - 1,)),
            pltpu.SemaphoreType.DMA((B,)),
            pltpu.VMEM((B, SQ, D), jnp.bfloat16),
        ],
        compiler_params=pltpu.CompilerParams(
            collective_id=0,
            vmem_limit_bytes=100 * 1024 * 1024,
        ),
    )(x, Wq, Wo, K_ext, V_ext)
